# baseline (speedup 1.0000x reference)
import functools

import jax
import jax.numpy as jnp
from jax import lax
from jax.experimental import pallas as pl
from jax.experimental.pallas import tpu as pltpu
from jax.experimental.pallas import tpu_sc as plsc

_NC = 2
_NS = 16
_NW = _NC * _NS


def _make_gather(n, da, dr):
    half = n // 2
    bpw = half // _NW
    ch = 512
    grp = bpw // ch
    k = ch // 128

    mesh = plsc.VectorSubcoreMesh(core_axis_name="c", subcore_axis_name="s")

    @functools.partial(
        pl.kernel,
        mesh=mesh,
        compiler_params=pltpu.CompilerParams(use_tc_tiling_on_sc=False),
        out_type=jax.ShapeDtypeStruct((half, 128), jnp.float32),
        scratch_types=[
            pltpu.VMEM((ch,), jnp.int32),
            pltpu.VMEM((ch,), jnp.int32),
            pltpu.VMEM((ch,), jnp.int32),
            pltpu.VMEM((ch,), jnp.int32),
            pltpu.VMEM((ch, da), jnp.float32),
            pltpu.VMEM((ch, da), jnp.float32),
            pltpu.VMEM((ch, dr), jnp.float32),
            pltpu.VMEM((ch, dr), jnp.float32),
            pltpu.SemaphoreType.DMA,
            pltpu.SemaphoreType.DMA,
        ],
    )
    def gather_k(aid_hbm, rid_hbm, at_hbm, rt_hbm, out,
                 aidx1, aidx2, ridx1, ridx2, a1, a2, r1, r2, sema, semr):
        wid = lax.axis_index("s") * _NC + lax.axis_index("c")

        def body(g, carry):
            base = pl.multiple_of(wid * bpw + g * ch, ch)
            base2 = base + half
            pltpu.sync_copy(aid_hbm.at[pl.ds(base, ch)], aidx1)
            pltpu.sync_copy(aid_hbm.at[pl.ds(base2, ch)], aidx2)
            pltpu.sync_copy(rid_hbm.at[pl.ds(base, ch)], ridx1)
            pltpu.sync_copy(rid_hbm.at[pl.ds(base2, ch)], ridx2)
            handles = []
            for j in range(k):
                sl = pl.ds(j * 128, 128)
                handles.append(pltpu.async_copy(
                    at_hbm.at[aidx1.at[sl]], a1.at[sl], sema))
                handles.append(pltpu.async_copy(
                    at_hbm.at[aidx2.at[sl]], a2.at[sl], sema))
                handles.append(pltpu.async_copy(
                    rt_hbm.at[ridx1.at[sl]], r1.at[sl], semr))
                handles.append(pltpu.async_copy(
                    rt_hbm.at[ridx2.at[sl]], r2.at[sl], semr))
            for h in handles:
                h.wait()
            rows = pl.ds(base, ch)
            pltpu.sync_copy(a1, out.at[rows, pl.ds(0, da)])
            pltpu.sync_copy(r1, out.at[rows, pl.ds(da, dr)])
            pltpu.sync_copy(a2, out.at[rows, pl.ds(64, da)])
            pltpu.sync_copy(r2, out.at[rows, pl.ds(64 + da, dr)])
            return carry

        lax.fori_loop(0, grp, body, 0)

    return gather_k


def _dense_body(x_ref, f1_ref, f2_ref, wc_ref, wnf_ref, sh4_ref, sh_ref,
                hh_ref, m1_ref, m2_ref, g2c_ref, o_ref, *, total_in, nd, dm):
    x = x_ref[...]
    rblk = x.shape[0]
    nfd = total_in - nd
    inv_n = 1.0 / float(total_in)
    inv_dm = 1.0 / float(dm)
    lane = lax.broadcasted_iota(jnp.int32, (1, 128), 1)
    memb = jnp.logical_or(lane < nd,
                          jnp.logical_and(lane >= 64, lane < 64 + nd))
    xf = jnp.where(memb, x, 0.0)

    zp = jnp.zeros((8 - nfd, rblk), jnp.float32)
    nf1 = jnp.concatenate(
        [jnp.log1p(jnp.maximum(f1_ref[...], 0.0)), zp], axis=0)
    nf2 = jnp.concatenate(
        [jnp.log1p(jnp.maximum(f2_ref[...], 0.0)), zp], axis=0)
    dn = (((0,), (0,)), ((), ()))

    sh = sh_ref[...]
    xs = jnp.dot(xf, sh, preferred_element_type=jnp.float32)
    x2s = jnp.dot(xf * xf, sh, preferred_element_type=jnp.float32)
    nfq = jnp.concatenate([nf1, nf2, nf1 * nf1, nf2 * nf2], axis=0)
    sums = lax.dot_general(nfq, sh4_ref[...], dn,
                           preferred_element_type=jnp.float32)
    mu1 = (xs[:, 0:1] + sums[:, 0:1]) * inv_n
    mu2 = (xs[:, 1:2] + sums[:, 1:2]) * inv_n
    var1 = (x2s[:, 0:1] + sums[:, 2:3]) * inv_n - mu1 * mu1
    var2 = (x2s[:, 1:2] + sums[:, 3:4]) * inv_n - mu2 * mu2
    inv1 = lax.rsqrt(jnp.maximum(var1, 0.0) + 1e-5)
    inv2 = lax.rsqrt(jnp.maximum(var2, 0.0) + 1e-5)

    y = jnp.dot(xf, wc_ref[...], preferred_element_type=jnp.float32)
    y = y + lax.dot_general(nfq, wnf_ref[...], dn,
                            preferred_element_type=jnp.float32)
    ones = jnp.ones((rblk, 1), jnp.float32)
    zp5 = jnp.zeros((rblk, 5), jnp.float32)
    p = jnp.concatenate([mu1 * inv1, mu2 * inv2, ones, zp5], axis=1)
    t = jnp.dot(p, m1_ref[...], preferred_element_type=jnp.float32)
    z1 = y[:, 0:dm] * inv1 + t[:, 0:dm]
    z2 = y[:, dm:2 * dm] * inv2 + t[:, dm:2 * dm]
    z = jnp.concatenate([z1, z2], axis=1)

    yg = 0.5 * z * (1.0 + lax.erf(z * 0.7071067811865476))
    hh = hh_ref[...]
    s = jnp.dot(yg, hh, preferred_element_type=jnp.float32)
    s2 = jnp.dot(yg * yg, hh, preferred_element_type=jnp.float32)
    mua = s[:, 0:1] * inv_dm
    mub = s[:, 1:2] * inv_dm
    vara = s2[:, 0:1] * inv_dm - mua * mua
    varb = s2[:, 1:2] * inv_dm - mub * mub
    ra = lax.rsqrt(jnp.maximum(vara, 0.0) + 1e-5)
    rb = lax.rsqrt(jnp.maximum(varb, 0.0) + 1e-5)
    p2 = jnp.concatenate([mua * ra, mub * rb, ones, zp5], axis=1)
    t2 = jnp.dot(p2, m2_ref[...], preferred_element_type=jnp.float32)
    g2c = g2c_ref[...]
    o_ref[0] = yg[:, 0:dm] * ra * g2c[:, 0:dm] + t2[:, 0:dm]
    o_ref[1] = yg[:, dm:2 * dm] * rb * g2c[:, dm:2 * dm] + t2[:, dm:2 * dm]


def _dense(packed, nft, wc, wnf, sh4, sh, hh, m1, m2, g2c, total_in, nd):
    half = packed.shape[0]
    nfd = nft.shape[0]
    dm = wc.shape[1] // 2
    rblk = 512
    nblk = half // rblk
    out = pl.pallas_call(
        functools.partial(_dense_body, total_in=total_in, nd=nd, dm=dm),
        grid=(nblk,),
        in_specs=[
            pl.BlockSpec((rblk, 128), lambda i: (i, 0)),
            pl.BlockSpec((nfd, rblk), lambda i: (0, i)),
            pl.BlockSpec((nfd, rblk), lambda i, _n=nblk: (0, i + _n)),
            pl.BlockSpec((128, 2 * dm), lambda i: (0, 0)),
            pl.BlockSpec((32, 2 * dm), lambda i: (0, 0)),
            pl.BlockSpec((32, 8), lambda i: (0, 0)),
            pl.BlockSpec((128, 8), lambda i: (0, 0)),
            pl.BlockSpec((2 * dm, 8), lambda i: (0, 0)),
            pl.BlockSpec((8, 2 * dm), lambda i: (0, 0)),
            pl.BlockSpec((8, 2 * dm), lambda i: (0, 0)),
            pl.BlockSpec((1, 2 * dm), lambda i: (0, 0)),
        ],
        out_specs=pl.BlockSpec((2, rblk, dm), lambda i: (0, i, 0)),
        out_shape=jax.ShapeDtypeStruct((2, half, dm), jnp.float32),
    )(packed, nft, nft, wc, wnf, sh4, sh, hh, m1, m2, g2c)
    return out.reshape(2 * half, dm)


def kernel(act_ids, res_ids, num_feats, act_table, res_table,
           ln1_g, ln1_b, W, b, ln2_g, ln2_b):
    n = act_ids.shape[0]
    da = act_table.shape[1]
    dr = res_table.shape[1]
    nfd = num_feats.shape[1]
    dm = W.shape[1]
    nd = da + dr
    tin = nd + nfd

    aid = act_ids.astype(jnp.int32)
    rid = res_ids.astype(jnp.int32)
    packed = _make_gather(n, da, dr)(aid, rid, act_table, res_table)

    wg = W * ln1_g[:, None]
    wc = jnp.zeros((128, 2 * dm), jnp.float32)
    wc = wc.at[0:nd, 0:dm].set(wg[0:nd])
    wc = wc.at[64:64 + nd, dm:2 * dm].set(wg[0:nd])
    wnf = jnp.zeros((32, 2 * dm), jnp.float32)
    wnf = wnf.at[0:nfd, 0:dm].set(wg[nd:])
    wnf = wnf.at[8:8 + nfd, dm:2 * dm].set(wg[nd:])
    sh4 = jnp.zeros((32, 8), jnp.float32)
    sh4 = sh4.at[0:8, 0].set(1.0)
    sh4 = sh4.at[8:16, 1].set(1.0)
    sh4 = sh4.at[16:24, 2].set(1.0)
    sh4 = sh4.at[24:32, 3].set(1.0)
    sh = jnp.zeros((128, 8), jnp.float32)
    sh = sh.at[0:nd, 0].set(1.0)
    sh = sh.at[64:64 + nd, 1].set(1.0)
    hh = jnp.zeros((2 * dm, 8), jnp.float32)
    hh = hh.at[0:dm, 0].set(1.0)
    hh = hh.at[dm:2 * dm, 1].set(1.0)
    csum = jnp.sum(wg, axis=0)
    beff = ln1_b @ W + b
    m1 = jnp.zeros((8, 2 * dm), jnp.float32)
    m1 = m1.at[0, 0:dm].set(-csum)
    m1 = m1.at[1, dm:2 * dm].set(-csum)
    m1 = m1.at[2, 0:dm].set(beff)
    m1 = m1.at[2, dm:2 * dm].set(beff)
    m2 = jnp.zeros((8, 2 * dm), jnp.float32)
    m2 = m2.at[0, 0:dm].set(-ln2_g)
    m2 = m2.at[1, dm:2 * dm].set(-ln2_g)
    m2 = m2.at[2, 0:dm].set(ln2_b)
    m2 = m2.at[2, dm:2 * dm].set(ln2_b)
    g2c = jnp.concatenate([ln2_g, ln2_g]).reshape(1, 2 * dm)
    nft = num_feats.T
    return _dense(packed, nft, wc, wnf, sh4, sh, hh, m1, m2, g2c, tin, nd)


# speedup vs baseline: 1.2200x; 1.2200x over previous
import functools

import jax
import jax.numpy as jnp
from jax import lax
from jax.experimental import pallas as pl
from jax.experimental.pallas import tpu as pltpu
from jax.experimental.pallas import tpu_sc as plsc

_NC = 2
_NS = 16
_NW = _NC * _NS


def _make_gather(n, da, dr):
    half = n // 2
    bpw = half // _NW
    ch = 512
    grp = bpw // ch
    k = ch // 128

    mesh = plsc.VectorSubcoreMesh(core_axis_name="c", subcore_axis_name="s")

    @functools.partial(
        pl.kernel,
        mesh=mesh,
        compiler_params=pltpu.CompilerParams(use_tc_tiling_on_sc=False),
        out_type=jax.ShapeDtypeStruct((half, 128), jnp.float32),
        scratch_types=[
            pltpu.VMEM((ch,), jnp.int32),
            pltpu.VMEM((ch,), jnp.int32),
            pltpu.VMEM((ch,), jnp.int32),
            pltpu.VMEM((ch,), jnp.int32),
            pltpu.VMEM((ch, da), jnp.float32),
            pltpu.VMEM((ch, da), jnp.float32),
            pltpu.VMEM((ch, dr), jnp.float32),
            pltpu.VMEM((ch, dr), jnp.float32),
            pltpu.SemaphoreType.DMA,
            pltpu.SemaphoreType.DMA,
        ],
    )
    def gather_k(aid_hbm, rid_hbm, at_hbm, rt_hbm, out,
                 aidx1, aidx2, ridx1, ridx2, a1, a2, r1, r2, sema, semr):
        wid = lax.axis_index("s") * _NC + lax.axis_index("c")

        def body(g, carry):
            base = pl.multiple_of(wid * bpw + g * ch, ch)
            base2 = base + half
            pltpu.sync_copy(aid_hbm.at[pl.ds(base, ch)], aidx1)
            pltpu.sync_copy(aid_hbm.at[pl.ds(base2, ch)], aidx2)
            pltpu.sync_copy(rid_hbm.at[pl.ds(base, ch)], ridx1)
            pltpu.sync_copy(rid_hbm.at[pl.ds(base2, ch)], ridx2)
            handles = []
            for j in range(k):
                sl = pl.ds(j * 128, 128)
                handles.append(pltpu.async_copy(
                    at_hbm.at[aidx1.at[sl]], a1.at[sl], sema))
                handles.append(pltpu.async_copy(
                    at_hbm.at[aidx2.at[sl]], a2.at[sl], sema))
                handles.append(pltpu.async_copy(
                    rt_hbm.at[ridx1.at[sl]], r1.at[sl], semr))
                handles.append(pltpu.async_copy(
                    rt_hbm.at[ridx2.at[sl]], r2.at[sl], semr))
            for h in handles:
                h.wait()
            rows = pl.ds(base, ch)
            pltpu.sync_copy(a1, out.at[rows, pl.ds(0, da)])
            pltpu.sync_copy(r1, out.at[rows, pl.ds(da, dr)])
            pltpu.sync_copy(a2, out.at[rows, pl.ds(64, da)])
            pltpu.sync_copy(r2, out.at[rows, pl.ds(64 + da, dr)])
            return carry

        lax.fori_loop(0, grp, body, 0)

    return gather_k


def _dense_body(x_ref, f1_ref, f2_ref, wc_ref, wnf_ref, sh4_ref, sh_ref,
                hh_ref, m1_ref, m2_ref, g2c_ref, o_ref, *, total_in, nd, dm):
    x = x_ref[...]
    rblk = x.shape[0]
    nfd = total_in - nd
    inv_n = 1.0 / float(total_in)
    inv_dm = 1.0 / float(dm)
    lane = lax.broadcasted_iota(jnp.int32, (1, 128), 1)
    memb = jnp.logical_or(lane < nd,
                          jnp.logical_and(lane >= 64, lane < 64 + nd))
    xf = jnp.where(memb, x, 0.0)

    zp = jnp.zeros((8 - nfd, rblk), jnp.float32)
    nf1 = jnp.concatenate(
        [jnp.log1p(jnp.maximum(f1_ref[...], 0.0)), zp], axis=0)
    nf2 = jnp.concatenate(
        [jnp.log1p(jnp.maximum(f2_ref[...], 0.0)), zp], axis=0)
    dn = (((0,), (0,)), ((), ()))

    sh = sh_ref[...]
    xs = jnp.dot(xf, sh, preferred_element_type=jnp.float32)
    x2s = jnp.dot(xf * xf, sh, preferred_element_type=jnp.float32)
    nfq = jnp.concatenate([nf1, nf2, nf1 * nf1, nf2 * nf2], axis=0)
    sums = lax.dot_general(nfq, sh4_ref[...], dn,
                           preferred_element_type=jnp.float32)
    mu1 = (xs[:, 0:1] + sums[:, 0:1]) * inv_n
    mu2 = (xs[:, 1:2] + sums[:, 1:2]) * inv_n
    var1 = (x2s[:, 0:1] + sums[:, 2:3]) * inv_n - mu1 * mu1
    var2 = (x2s[:, 1:2] + sums[:, 3:4]) * inv_n - mu2 * mu2
    inv1 = lax.rsqrt(jnp.maximum(var1, 0.0) + 1e-5)
    inv2 = lax.rsqrt(jnp.maximum(var2, 0.0) + 1e-5)

    y = jnp.dot(xf, wc_ref[...], preferred_element_type=jnp.float32)
    y = y + lax.dot_general(nfq, wnf_ref[...], dn,
                            preferred_element_type=jnp.float32)
    ones = jnp.ones((rblk, 1), jnp.float32)
    zp5 = jnp.zeros((rblk, 5), jnp.float32)
    p = jnp.concatenate([mu1 * inv1, mu2 * inv2, ones, zp5], axis=1)
    t = jnp.dot(p, m1_ref[...], preferred_element_type=jnp.float32)
    z1 = y[:, 0:dm] * inv1 + t[:, 0:dm]
    z2 = y[:, dm:2 * dm] * inv2 + t[:, dm:2 * dm]
    z = jnp.concatenate([z1, z2], axis=1)

    yg = 0.5 * z * (1.0 + lax.erf(z * 0.7071067811865476))
    hh = hh_ref[...]
    s = jnp.dot(yg, hh, preferred_element_type=jnp.float32)
    s2 = jnp.dot(yg * yg, hh, preferred_element_type=jnp.float32)
    mua = s[:, 0:1] * inv_dm
    mub = s[:, 1:2] * inv_dm
    vara = s2[:, 0:1] * inv_dm - mua * mua
    varb = s2[:, 1:2] * inv_dm - mub * mub
    ra = lax.rsqrt(jnp.maximum(vara, 0.0) + 1e-5)
    rb = lax.rsqrt(jnp.maximum(varb, 0.0) + 1e-5)
    p2 = jnp.concatenate([mua * ra, mub * rb, ones, zp5], axis=1)
    t2 = jnp.dot(p2, m2_ref[...], preferred_element_type=jnp.float32)
    g2c = g2c_ref[...]
    o_ref[0] = yg[:, 0:dm] * ra * g2c[:, 0:dm] + t2[:, 0:dm]
    o_ref[1] = yg[:, dm:2 * dm] * rb * g2c[:, dm:2 * dm] + t2[:, dm:2 * dm]


def _dense(packed, nft, wc, wnf, sh4, sh, hh, m1, m2, g2c, total_in, nd):
    half = packed.shape[0]
    nfd = nft.shape[0]
    dm = wc.shape[1] // 2
    rblk = 2048
    nblk = half // rblk
    out = pl.pallas_call(
        functools.partial(_dense_body, total_in=total_in, nd=nd, dm=dm),
        grid=(nblk,),
        in_specs=[
            pl.BlockSpec((rblk, 128), lambda i: (i, 0)),
            pl.BlockSpec((nfd, rblk), lambda i: (0, i)),
            pl.BlockSpec((nfd, rblk), lambda i, _n=nblk: (0, i + _n)),
            pl.BlockSpec((128, 2 * dm), lambda i: (0, 0)),
            pl.BlockSpec((32, 2 * dm), lambda i: (0, 0)),
            pl.BlockSpec((32, 8), lambda i: (0, 0)),
            pl.BlockSpec((128, 8), lambda i: (0, 0)),
            pl.BlockSpec((2 * dm, 8), lambda i: (0, 0)),
            pl.BlockSpec((8, 2 * dm), lambda i: (0, 0)),
            pl.BlockSpec((8, 2 * dm), lambda i: (0, 0)),
            pl.BlockSpec((1, 2 * dm), lambda i: (0, 0)),
        ],
        out_specs=pl.BlockSpec((2, rblk, dm), lambda i: (0, i, 0)),
        out_shape=jax.ShapeDtypeStruct((2, half, dm), jnp.float32),
    )(packed, nft, nft, wc, wnf, sh4, sh, hh, m1, m2, g2c)
    return out.reshape(2 * half, dm)


def kernel(act_ids, res_ids, num_feats, act_table, res_table,
           ln1_g, ln1_b, W, b, ln2_g, ln2_b):
    n = act_ids.shape[0]
    da = act_table.shape[1]
    dr = res_table.shape[1]
    nfd = num_feats.shape[1]
    dm = W.shape[1]
    nd = da + dr
    tin = nd + nfd

    aid = act_ids.astype(jnp.int32)
    rid = res_ids.astype(jnp.int32)
    packed = _make_gather(n, da, dr)(aid, rid, act_table, res_table)

    wg = W * ln1_g[:, None]
    wc = jnp.zeros((128, 2 * dm), jnp.float32)
    wc = wc.at[0:nd, 0:dm].set(wg[0:nd])
    wc = wc.at[64:64 + nd, dm:2 * dm].set(wg[0:nd])
    wnf = jnp.zeros((32, 2 * dm), jnp.float32)
    wnf = wnf.at[0:nfd, 0:dm].set(wg[nd:])
    wnf = wnf.at[8:8 + nfd, dm:2 * dm].set(wg[nd:])
    sh4 = jnp.zeros((32, 8), jnp.float32)
    sh4 = sh4.at[0:8, 0].set(1.0)
    sh4 = sh4.at[8:16, 1].set(1.0)
    sh4 = sh4.at[16:24, 2].set(1.0)
    sh4 = sh4.at[24:32, 3].set(1.0)
    sh = jnp.zeros((128, 8), jnp.float32)
    sh = sh.at[0:nd, 0].set(1.0)
    sh = sh.at[64:64 + nd, 1].set(1.0)
    hh = jnp.zeros((2 * dm, 8), jnp.float32)
    hh = hh.at[0:dm, 0].set(1.0)
    hh = hh.at[dm:2 * dm, 1].set(1.0)
    csum = jnp.sum(wg, axis=0)
    beff = ln1_b @ W + b
    m1 = jnp.zeros((8, 2 * dm), jnp.float32)
    m1 = m1.at[0, 0:dm].set(-csum)
    m1 = m1.at[1, dm:2 * dm].set(-csum)
    m1 = m1.at[2, 0:dm].set(beff)
    m1 = m1.at[2, dm:2 * dm].set(beff)
    m2 = jnp.zeros((8, 2 * dm), jnp.float32)
    m2 = m2.at[0, 0:dm].set(-ln2_g)
    m2 = m2.at[1, dm:2 * dm].set(-ln2_g)
    m2 = m2.at[2, 0:dm].set(ln2_b)
    m2 = m2.at[2, dm:2 * dm].set(ln2_b)
    g2c = jnp.concatenate([ln2_g, ln2_g]).reshape(1, 2 * dm)
    nft = num_feats.T
    return _dense(packed, nft, wc, wnf, sh4, sh, hh, m1, m2, g2c, tin, nd)


# speedup vs baseline: 1.2326x; 1.0103x over previous
import functools

import jax
import jax.numpy as jnp
from jax import lax
from jax.experimental import pallas as pl
from jax.experimental.pallas import tpu as pltpu
from jax.experimental.pallas import tpu_sc as plsc

_NC = 2
_NS = 16
_NW = _NC * _NS


def _make_gather(n, da, dr):
    half = n // 2
    bpw = half // _NW
    ch = 512
    grp = bpw // ch
    k = ch // 128

    mesh = plsc.VectorSubcoreMesh(core_axis_name="c", subcore_axis_name="s")

    @functools.partial(
        pl.kernel,
        mesh=mesh,
        compiler_params=pltpu.CompilerParams(use_tc_tiling_on_sc=False),
        out_type=jax.ShapeDtypeStruct((half, 128), jnp.float32),
        scratch_types=[
            pltpu.VMEM((ch,), jnp.int32),
            pltpu.VMEM((ch,), jnp.int32),
            pltpu.VMEM((ch,), jnp.int32),
            pltpu.VMEM((ch,), jnp.int32),
            pltpu.VMEM((ch, da), jnp.float32),
            pltpu.VMEM((ch, da), jnp.float32),
            pltpu.VMEM((ch, dr), jnp.float32),
            pltpu.VMEM((ch, dr), jnp.float32),
            pltpu.SemaphoreType.DMA,
            pltpu.SemaphoreType.DMA,
        ],
    )
    def gather_k(aid_hbm, rid_hbm, at_hbm, rt_hbm, out,
                 aidx1, aidx2, ridx1, ridx2, a1, a2, r1, r2, sema, semr):
        wid = lax.axis_index("s") * _NC + lax.axis_index("c")

        def body(g, carry):
            base = pl.multiple_of(wid * bpw + g * ch, ch)
            base2 = base + half
            pltpu.sync_copy(aid_hbm.at[pl.ds(base, ch)], aidx1)
            pltpu.sync_copy(aid_hbm.at[pl.ds(base2, ch)], aidx2)
            pltpu.sync_copy(rid_hbm.at[pl.ds(base, ch)], ridx1)
            pltpu.sync_copy(rid_hbm.at[pl.ds(base2, ch)], ridx2)
            handles = []
            for j in range(k):
                sl = pl.ds(j * 128, 128)
                handles.append(pltpu.async_copy(
                    at_hbm.at[aidx1.at[sl]], a1.at[sl], sema))
                handles.append(pltpu.async_copy(
                    at_hbm.at[aidx2.at[sl]], a2.at[sl], sema))
                handles.append(pltpu.async_copy(
                    rt_hbm.at[ridx1.at[sl]], r1.at[sl], semr))
                handles.append(pltpu.async_copy(
                    rt_hbm.at[ridx2.at[sl]], r2.at[sl], semr))
            for h in handles:
                h.wait()
            rows = pl.ds(base, ch)
            pltpu.sync_copy(a1, out.at[rows, pl.ds(0, da)])
            pltpu.sync_copy(r1, out.at[rows, pl.ds(da, dr)])
            pltpu.sync_copy(a2, out.at[rows, pl.ds(64, da)])
            pltpu.sync_copy(r2, out.at[rows, pl.ds(64 + da, dr)])
            return carry

        lax.fori_loop(0, grp, body, 0)

    return gather_k


def _dense_body(x_ref, f1_ref, f2_ref, wc_ref, wnf_ref, sh4_ref, sh_ref,
                hh_ref, m1_ref, m2_ref, g2c_ref, o_ref, *, total_in, nd, dm):
    x = x_ref[...]
    rblk = x.shape[0]
    nfd = total_in - nd
    inv_n = 1.0 / float(total_in)
    inv_dm = 1.0 / float(dm)
    lane = lax.broadcasted_iota(jnp.int32, (1, 128), 1)
    memb = jnp.logical_or(lane < nd,
                          jnp.logical_and(lane >= 64, lane < 64 + nd))
    xf = jnp.where(memb, x, 0.0)

    zp = jnp.zeros((8 - nfd, rblk), jnp.float32)
    nf1 = jnp.concatenate(
        [jnp.log1p(jnp.maximum(f1_ref[...], 0.0)), zp], axis=0)
    nf2 = jnp.concatenate(
        [jnp.log1p(jnp.maximum(f2_ref[...], 0.0)), zp], axis=0)
    dn = (((0,), (0,)), ((), ()))

    sh = sh_ref[...]
    xs = jnp.dot(xf, sh, preferred_element_type=jnp.float32)
    x2s = jnp.dot(xf * xf, sh, preferred_element_type=jnp.float32)
    nfq = jnp.concatenate([nf1, nf2, nf1 * nf1, nf2 * nf2], axis=0)
    sums = lax.dot_general(nfq, sh4_ref[...], dn,
                           preferred_element_type=jnp.float32)
    mu1 = (xs[:, 0:1] + sums[:, 0:1]) * inv_n
    mu2 = (xs[:, 1:2] + sums[:, 1:2]) * inv_n
    var1 = (x2s[:, 0:1] + sums[:, 2:3]) * inv_n - mu1 * mu1
    var2 = (x2s[:, 1:2] + sums[:, 3:4]) * inv_n - mu2 * mu2
    inv1 = lax.rsqrt(jnp.maximum(var1, 0.0) + 1e-5)
    inv2 = lax.rsqrt(jnp.maximum(var2, 0.0) + 1e-5)

    y = jnp.dot(xf, wc_ref[...], preferred_element_type=jnp.float32)
    y = y + lax.dot_general(nfq, wnf_ref[...], dn,
                            preferred_element_type=jnp.float32)
    ones = jnp.ones((rblk, 1), jnp.float32)
    zp5 = jnp.zeros((rblk, 5), jnp.float32)
    p = jnp.concatenate([mu1 * inv1, mu2 * inv2, ones, zp5], axis=1)
    t = jnp.dot(p, m1_ref[...], preferred_element_type=jnp.float32)
    z1 = y[:, 0:dm] * inv1 + t[:, 0:dm]
    z2 = y[:, dm:2 * dm] * inv2 + t[:, dm:2 * dm]
    z = jnp.concatenate([z1, z2], axis=1)

    yg = 0.5 * z * (1.0 + lax.erf(z * 0.7071067811865476))
    hh = hh_ref[...]
    s = jnp.dot(yg, hh, preferred_element_type=jnp.float32)
    s2 = jnp.dot(yg * yg, hh, preferred_element_type=jnp.float32)
    mua = s[:, 0:1] * inv_dm
    mub = s[:, 1:2] * inv_dm
    vara = s2[:, 0:1] * inv_dm - mua * mua
    varb = s2[:, 1:2] * inv_dm - mub * mub
    ra = lax.rsqrt(jnp.maximum(vara, 0.0) + 1e-5)
    rb = lax.rsqrt(jnp.maximum(varb, 0.0) + 1e-5)
    p2 = jnp.concatenate([mua * ra, mub * rb, ones, zp5], axis=1)
    t2 = jnp.dot(p2, m2_ref[...], preferred_element_type=jnp.float32)
    g2c = g2c_ref[...]
    o_ref[0] = yg[:, 0:dm] * ra * g2c[:, 0:dm] + t2[:, 0:dm]
    o_ref[1] = yg[:, dm:2 * dm] * rb * g2c[:, dm:2 * dm] + t2[:, dm:2 * dm]


def _dense(packed, nft, wc, wnf, sh4, sh, hh, m1, m2, g2c, total_in, nd):
    half = packed.shape[0]
    nfd = nft.shape[0]
    dm = wc.shape[1] // 2
    rblk = 4096
    nblk = half // rblk
    out = pl.pallas_call(
        functools.partial(_dense_body, total_in=total_in, nd=nd, dm=dm),
        grid=(nblk,),
        in_specs=[
            pl.BlockSpec((rblk, 128), lambda i: (i, 0)),
            pl.BlockSpec((nfd, rblk), lambda i: (0, i)),
            pl.BlockSpec((nfd, rblk), lambda i, _n=nblk: (0, i + _n)),
            pl.BlockSpec((128, 2 * dm), lambda i: (0, 0)),
            pl.BlockSpec((32, 2 * dm), lambda i: (0, 0)),
            pl.BlockSpec((32, 8), lambda i: (0, 0)),
            pl.BlockSpec((128, 8), lambda i: (0, 0)),
            pl.BlockSpec((2 * dm, 8), lambda i: (0, 0)),
            pl.BlockSpec((8, 2 * dm), lambda i: (0, 0)),
            pl.BlockSpec((8, 2 * dm), lambda i: (0, 0)),
            pl.BlockSpec((1, 2 * dm), lambda i: (0, 0)),
        ],
        out_specs=pl.BlockSpec((2, rblk, dm), lambda i: (0, i, 0)),
        out_shape=jax.ShapeDtypeStruct((2, half, dm), jnp.float32),
    )(packed, nft, nft, wc, wnf, sh4, sh, hh, m1, m2, g2c)
    return out.reshape(2 * half, dm)


def kernel(act_ids, res_ids, num_feats, act_table, res_table,
           ln1_g, ln1_b, W, b, ln2_g, ln2_b):
    n = act_ids.shape[0]
    da = act_table.shape[1]
    dr = res_table.shape[1]
    nfd = num_feats.shape[1]
    dm = W.shape[1]
    nd = da + dr
    tin = nd + nfd

    aid = act_ids.astype(jnp.int32)
    rid = res_ids.astype(jnp.int32)
    packed = _make_gather(n, da, dr)(aid, rid, act_table, res_table)

    wg = W * ln1_g[:, None]
    wc = jnp.zeros((128, 2 * dm), jnp.float32)
    wc = wc.at[0:nd, 0:dm].set(wg[0:nd])
    wc = wc.at[64:64 + nd, dm:2 * dm].set(wg[0:nd])
    wnf = jnp.zeros((32, 2 * dm), jnp.float32)
    wnf = wnf.at[0:nfd, 0:dm].set(wg[nd:])
    wnf = wnf.at[8:8 + nfd, dm:2 * dm].set(wg[nd:])
    sh4 = jnp.zeros((32, 8), jnp.float32)
    sh4 = sh4.at[0:8, 0].set(1.0)
    sh4 = sh4.at[8:16, 1].set(1.0)
    sh4 = sh4.at[16:24, 2].set(1.0)
    sh4 = sh4.at[24:32, 3].set(1.0)
    sh = jnp.zeros((128, 8), jnp.float32)
    sh = sh.at[0:nd, 0].set(1.0)
    sh = sh.at[64:64 + nd, 1].set(1.0)
    hh = jnp.zeros((2 * dm, 8), jnp.float32)
    hh = hh.at[0:dm, 0].set(1.0)
    hh = hh.at[dm:2 * dm, 1].set(1.0)
    csum = jnp.sum(wg, axis=0)
    beff = ln1_b @ W + b
    m1 = jnp.zeros((8, 2 * dm), jnp.float32)
    m1 = m1.at[0, 0:dm].set(-csum)
    m1 = m1.at[1, dm:2 * dm].set(-csum)
    m1 = m1.at[2, 0:dm].set(beff)
    m1 = m1.at[2, dm:2 * dm].set(beff)
    m2 = jnp.zeros((8, 2 * dm), jnp.float32)
    m2 = m2.at[0, 0:dm].set(-ln2_g)
    m2 = m2.at[1, dm:2 * dm].set(-ln2_g)
    m2 = m2.at[2, 0:dm].set(ln2_b)
    m2 = m2.at[2, dm:2 * dm].set(ln2_b)
    g2c = jnp.concatenate([ln2_g, ln2_g]).reshape(1, 2 * dm)
    nft = num_feats.T
    return _dense(packed, nft, wc, wnf, sh4, sh, hh, m1, m2, g2c, tin, nd)
